# Initial kernel scaffold; baseline (speedup 1.0000x reference)
import jax
import jax.numpy as jnp
from jax.experimental import pallas as pl
from jax.experimental.pallas import tpu as pltpu

N = 10000
D = 128
G = 64
N_CLASS = 10
BM1 = 400
BM2 = 2000
NI1 = N // BM1
NI2 = N // BM2
F32 = jnp.float32


def _pass1_body(adj_ref, x_ref, w1_ref, b1_ref, idx_ref,
                p_ref, asum_ref, y1_s):
    i = pl.program_id(0)

    @pl.when(i == 0)
    def _init():
        y1_s[...] = (jnp.dot(x_ref[...], w1_ref[...],
                             preferred_element_type=F32) + b1_ref[...])
        asum_ref[...] = jnp.zeros_like(asum_ref)

    adj = adj_ref[...]
    p_ref[...] = jnp.dot(adj, y1_s[...], preferred_element_type=F32)
    seg = jax.lax.broadcasted_iota(jnp.int32, (G, BM1), 0)
    s_blk = (seg == idx_ref[0]).astype(F32)
    asum_ref[...] += jnp.dot(s_blk, adj, preferred_element_type=F32)


def _pass2_body(p_ref, x_ref, asum_ref, idx_ref, w1_ref, b1_ref,
                w2_ref, b2_ref, w3_ref, b3_ref, w4_ref, b4_ref,
                gamma_ref, beta_ref, out_ref, pool_s):
    k = pl.program_id(0)

    @pl.when(k == 0)
    def _init():
        pool_s[...] = jnp.zeros_like(pool_s)

    y1 = (jnp.dot(x_ref[...], w1_ref[...], preferred_element_type=F32)
          + b1_ref[...])
    h1 = jnp.maximum(p_ref[...] + y1, 0.0)
    z2 = (jnp.dot(h1, w2_ref[...], preferred_element_type=F32)
          + b2_ref[...])
    seg = jax.lax.broadcasted_iota(jnp.int32, (G, BM2), 0)
    t = asum_ref[...] + (seg == idx_ref[0]).astype(F32)
    pool_s[...] += jnp.dot(t, z2, preferred_element_type=F32)

    @pl.when(k == NI2 - 1)
    def _head():
        inv = 1.0 / jnp.sqrt(jnp.float32(1.0 + 1e-5))
        o = gamma_ref[...] * (pool_s[...] * inv) + beta_ref[...]
        o = jnp.maximum(jnp.dot(o, w3_ref[...], preferred_element_type=F32)
                        + b3_ref[...], 0.0)
        logits = (jnp.dot(o, w4_ref[...], preferred_element_type=F32)
                  + b4_ref[...])
        m = jnp.max(logits, axis=1, keepdims=True)
        lse = jnp.log(jnp.sum(jnp.exp(logits - m), axis=1, keepdims=True))
        out_ref[...] = logits - m - lse


def kernel(x_in, adj, idx, W1, b1, W2, b2, W3, b3, W4, b4, gamma, beta,
           interpret=False):
    idx1 = idx.reshape(NI1, 1, BM1)
    idx2 = idx.reshape(NI2, 1, BM2)
    b1r = b1.reshape(1, D)
    b2r = b2.reshape(1, D)
    b3r = b3.reshape(1, D)
    b4r = b4.reshape(1, N_CLASS)
    gammar = gamma.reshape(1, D)
    betar = beta.reshape(1, D)

    p, asum = pl.pallas_call(
        _pass1_body,
        grid=(NI1,),
        in_specs=[
            pl.BlockSpec((BM1, N), lambda i: (i, 0)),
            pl.BlockSpec((N, D), lambda i: (0, 0)),
            pl.BlockSpec((D, D), lambda i: (0, 0)),
            pl.BlockSpec((1, D), lambda i: (0, 0)),
            pl.BlockSpec((1, 1, BM1), lambda i: (i, 0, 0)),
        ],
        out_specs=[
            pl.BlockSpec((BM1, D), lambda i: (i, 0)),
            pl.BlockSpec((G, N), lambda i: (0, 0)),
        ],
        out_shape=[
            jax.ShapeDtypeStruct((N, D), F32),
            jax.ShapeDtypeStruct((G, N), F32),
        ],
        scratch_shapes=[pltpu.VMEM((N, D), F32)],
        compiler_params=pltpu.CompilerParams(
            dimension_semantics=("arbitrary",)),
        interpret=interpret,
    )(adj, x_in, W1, b1r, idx1)

    out = pl.pallas_call(
        _pass2_body,
        grid=(NI2,),
        in_specs=[
            pl.BlockSpec((BM2, D), lambda k: (k, 0)),
            pl.BlockSpec((BM2, D), lambda k: (k, 0)),
            pl.BlockSpec((G, BM2), lambda k: (0, k)),
            pl.BlockSpec((1, 1, BM2), lambda k: (k, 0, 0)),
            pl.BlockSpec((D, D), lambda k: (0, 0)),
            pl.BlockSpec((1, D), lambda k: (0, 0)),
            pl.BlockSpec((D, D), lambda k: (0, 0)),
            pl.BlockSpec((1, D), lambda k: (0, 0)),
            pl.BlockSpec((D, D), lambda k: (0, 0)),
            pl.BlockSpec((1, D), lambda k: (0, 0)),
            pl.BlockSpec((D, N_CLASS), lambda k: (0, 0)),
            pl.BlockSpec((1, N_CLASS), lambda k: (0, 0)),
            pl.BlockSpec((1, D), lambda k: (0, 0)),
            pl.BlockSpec((1, D), lambda k: (0, 0)),
        ],
        out_specs=pl.BlockSpec((G, N_CLASS), lambda k: (0, 0)),
        out_shape=jax.ShapeDtypeStruct((G, N_CLASS), F32),
        scratch_shapes=[pltpu.VMEM((G, D), F32)],
        compiler_params=pltpu.CompilerParams(
            dimension_semantics=("arbitrary",)),
        interpret=interpret,
    )(p, x_in, asum, idx2, W1, b1r, W2, b2r, W3, b3r, W4, b4r,
      gammar, betar)

    return out


# speedup vs baseline: 2.4297x; 2.4297x over previous
import jax
import jax.numpy as jnp
from jax.experimental import pallas as pl
from jax.experimental.pallas import tpu as pltpu

N = 10000
D = 128
G = 64
N_CLASS = 10
BM1 = 400
BM2 = 2000
NI1 = N // BM1
NI2 = N // BM2
F32 = jnp.float32


def _pass1_body(adj_ref, x_ref, w1_ref, b1_ref, idx_ref,
                p_ref, asum_ref, y1_s):
    i = pl.program_id(0)

    @pl.when(i == 0)
    def _init():
        y1_s[...] = (jnp.dot(x_ref[...], w1_ref[...],
                             preferred_element_type=F32) + b1_ref[...])
        asum_ref[...] = jnp.zeros_like(asum_ref)

    adj = adj_ref[...]
    p_ref[...] = jnp.dot(adj, y1_s[...], preferred_element_type=F32)
    seg = jax.lax.broadcasted_iota(jnp.int32, (G, BM1), 0)
    s_blk = (seg == idx_ref[0]).astype(F32)
    asum_ref[...] += jnp.dot(s_blk, adj, preferred_element_type=F32)


def _pass2_body(p_ref, x_ref, asum_ref, idx_ref, w1_ref, b1_ref,
                w2_ref, b2_ref, w3_ref, b3_ref, w4_ref, b4_ref,
                gamma_ref, beta_ref, out_ref, pool_s, z2_s):
    k = pl.program_id(0)

    @pl.when(k == 0)
    def _init():
        pool_s[...] = jnp.zeros_like(pool_s)

    y1 = (jnp.dot(x_ref[...], w1_ref[...], preferred_element_type=F32)
          + b1_ref[...])
    h1 = jnp.maximum(p_ref[...] + y1, 0.0)
    z2 = (jnp.dot(h1, w2_ref[...], preferred_element_type=F32)
          + b2_ref[...])
    z2_s[pl.ds(k * BM2, BM2), :] = z2
    seg = jax.lax.broadcasted_iota(jnp.int32, (G, BM2), 0)
    s_blk = (seg == idx_ref[0]).astype(F32)
    pool_s[...] += jnp.dot(s_blk, z2, preferred_element_type=F32)

    @pl.when(k == NI2 - 1)
    def _head():
        inv = 1.0 / jnp.sqrt(jnp.float32(1.0 + 1e-5))
        pooled = pool_s[...] + jnp.dot(asum_ref[...], z2_s[...],
                                       preferred_element_type=F32)
        o = gamma_ref[...] * (pooled * inv) + beta_ref[...]
        o = jnp.maximum(jnp.dot(o, w3_ref[...], preferred_element_type=F32)
                        + b3_ref[...], 0.0)
        logits = (jnp.dot(o, w4_ref[...], preferred_element_type=F32)
                  + b4_ref[...])
        m = jnp.max(logits, axis=1, keepdims=True)
        lse = jnp.log(jnp.sum(jnp.exp(logits - m), axis=1, keepdims=True))
        out_ref[...] = logits - m - lse


def kernel(x_in, adj, idx, W1, b1, W2, b2, W3, b3, W4, b4, gamma, beta,
           interpret=False):
    idx1 = idx.reshape(NI1, 1, BM1)
    idx2 = idx.reshape(NI2, 1, BM2)
    b1r = b1.reshape(1, D)
    b2r = b2.reshape(1, D)
    b3r = b3.reshape(1, D)
    b4r = b4.reshape(1, N_CLASS)
    gammar = gamma.reshape(1, D)
    betar = beta.reshape(1, D)

    p, asum = pl.pallas_call(
        _pass1_body,
        grid=(NI1,),
        in_specs=[
            pl.BlockSpec((BM1, N), lambda i: (i, 0)),
            pl.BlockSpec((N, D), lambda i: (0, 0)),
            pl.BlockSpec((D, D), lambda i: (0, 0)),
            pl.BlockSpec((1, D), lambda i: (0, 0)),
            pl.BlockSpec((1, 1, BM1), lambda i: (i, 0, 0)),
        ],
        out_specs=[
            pl.BlockSpec((BM1, D), lambda i: (i, 0)),
            pl.BlockSpec((G, N), lambda i: (0, 0)),
        ],
        out_shape=[
            jax.ShapeDtypeStruct((N, D), F32),
            jax.ShapeDtypeStruct((G, N), F32),
        ],
        scratch_shapes=[pltpu.VMEM((N, D), F32)],
        compiler_params=pltpu.CompilerParams(
            dimension_semantics=("arbitrary",)),
        interpret=interpret,
    )(adj, x_in, W1, b1r, idx1)

    out = pl.pallas_call(
        _pass2_body,
        grid=(NI2,),
        in_specs=[
            pl.BlockSpec((BM2, D), lambda k: (k, 0)),
            pl.BlockSpec((BM2, D), lambda k: (k, 0)),
            pl.BlockSpec((G, N), lambda k: (0, 0)),
            pl.BlockSpec((1, 1, BM2), lambda k: (k, 0, 0)),
            pl.BlockSpec((D, D), lambda k: (0, 0)),
            pl.BlockSpec((1, D), lambda k: (0, 0)),
            pl.BlockSpec((D, D), lambda k: (0, 0)),
            pl.BlockSpec((1, D), lambda k: (0, 0)),
            pl.BlockSpec((D, D), lambda k: (0, 0)),
            pl.BlockSpec((1, D), lambda k: (0, 0)),
            pl.BlockSpec((D, N_CLASS), lambda k: (0, 0)),
            pl.BlockSpec((1, N_CLASS), lambda k: (0, 0)),
            pl.BlockSpec((1, D), lambda k: (0, 0)),
            pl.BlockSpec((1, D), lambda k: (0, 0)),
        ],
        out_specs=pl.BlockSpec((G, N_CLASS), lambda k: (0, 0)),
        out_shape=jax.ShapeDtypeStruct((G, N_CLASS), F32),
        scratch_shapes=[pltpu.VMEM((G, D), F32), pltpu.VMEM((N, D), F32)],
        compiler_params=pltpu.CompilerParams(
            dimension_semantics=("arbitrary",)),
        interpret=interpret,
    )(p, x_in, asum, idx2, W1, b1r, W2, b2r, W3, b3r, W4, b4r,
      gammar, betar)

    return out


# speedup vs baseline: 2.5822x; 1.0628x over previous
import jax
import jax.numpy as jnp
from jax.experimental import pallas as pl
from jax.experimental.pallas import tpu as pltpu

N = 10000
D = 128
G = 64
N_CLASS = 10
BM = 400
NI = N // BM
F32 = jnp.float32


def _body(adj_ref, x_ref, idx_ref, w1_ref, b1_ref, w2_ref, b2_ref,
          w3_ref, b3_ref, w4_ref, b4_ref, gamma_ref, beta_ref,
          out_ref, y1_s, z2_s, asum_s, pool_s):
    i = pl.program_id(0)

    @pl.when(i == 0)
    def _init():
        y1_s[...] = (jnp.dot(x_ref[...], w1_ref[...],
                             preferred_element_type=F32) + b1_ref[...])
        asum_s[...] = jnp.zeros_like(asum_s)
        pool_s[...] = jnp.zeros_like(pool_s)

    adj = adj_ref[...]
    seg = jax.lax.broadcasted_iota(jnp.int32, (G, BM), 0)
    s_blk = (seg == idx_ref[0]).astype(F32)
    asum_s[...] += jnp.dot(s_blk, adj, preferred_element_type=F32)

    p_blk = jnp.dot(adj, y1_s[...], preferred_element_type=F32)
    h1 = jnp.maximum(p_blk + y1_s[pl.ds(i * BM, BM), :], 0.0)
    z2 = (jnp.dot(h1, w2_ref[...], preferred_element_type=F32)
          + b2_ref[...])
    z2_s[pl.ds(i * BM, BM), :] = z2
    pool_s[...] += jnp.dot(s_blk, z2, preferred_element_type=F32)

    @pl.when(i == NI - 1)
    def _head():
        pooled = pool_s[...] + jnp.dot(asum_s[...], z2_s[...],
                                       preferred_element_type=F32)
        inv = 1.0 / jnp.sqrt(jnp.float32(1.0 + 1e-5))
        o = gamma_ref[...] * (pooled * inv) + beta_ref[...]
        o = jnp.maximum(jnp.dot(o, w3_ref[...], preferred_element_type=F32)
                        + b3_ref[...], 0.0)
        logits = (jnp.dot(o, w4_ref[...], preferred_element_type=F32)
                  + b4_ref[...])
        m = jnp.max(logits, axis=1, keepdims=True)
        lse = jnp.log(jnp.sum(jnp.exp(logits - m), axis=1, keepdims=True))
        out_ref[...] = logits - m - lse


def kernel(x_in, adj, idx, W1, b1, W2, b2, W3, b3, W4, b4, gamma, beta,
           interpret=False):
    idx3 = idx.reshape(NI, 1, BM)
    full = lambda shape: pl.BlockSpec(shape, lambda i: (0,) * len(shape))

    return pl.pallas_call(
        _body,
        grid=(NI,),
        in_specs=[
            pl.BlockSpec((BM, N), lambda i: (i, 0)),
            full((N, D)),
            pl.BlockSpec((1, 1, BM), lambda i: (i, 0, 0)),
            full((D, D)),
            full((1, D)),
            full((D, D)),
            full((1, D)),
            full((D, D)),
            full((1, D)),
            full((D, N_CLASS)),
            full((1, N_CLASS)),
            full((1, D)),
            full((1, D)),
        ],
        out_specs=full((G, N_CLASS)),
        out_shape=jax.ShapeDtypeStruct((G, N_CLASS), F32),
        scratch_shapes=[
            pltpu.VMEM((N, D), F32),
            pltpu.VMEM((N, D), F32),
            pltpu.VMEM((G, N), F32),
            pltpu.VMEM((G, D), F32),
        ],
        compiler_params=pltpu.CompilerParams(
            dimension_semantics=("arbitrary",)),
        interpret=interpret,
    )(adj, x_in, idx3, W1, b1.reshape(1, D), W2, b2.reshape(1, D),
      W3, b3.reshape(1, D), W4, b4.reshape(1, N_CLASS),
      gamma.reshape(1, D), beta.reshape(1, D))
